# baseline (speedup 1.0000x reference)
import functools

import jax
import jax.numpy as jnp
from jax import lax
from jax.experimental import pallas as pl
from jax.experimental.pallas import tpu as pltpu
from jax.experimental.pallas import tpu_sc as plsc

N, FD, ED, S, B = 10000, 128, 64, 32, 10000
NC, NS = 2, 16
NW = NC * NS
CH = 128
NH = 2
BH = B // NH
NCH = (BH * S) // CH
EPW = 40
NCHP = NW * EPW
NBUF = 4
QCH = 40
NQCH = BH // QCH
QPW = 8
NQCHP = 128
QBUF = 2
BB = 200
PROJ_BLK = 1000


def _proj_body(x_ref, wkv_ref, wmq_ref, bkv_ref, bmq_ref, pkv_ref, pmq_ref):
    x = x_ref[:]
    pkv_ref[:] = jnp.dot(x, wkv_ref[:], preferred_element_type=jnp.float32) + bkv_ref[:]
    pmq_ref[:] = jnp.dot(x, wmq_ref[:], preferred_element_type=jnp.float32) + bmq_ref[:]


def _project(id2feat, wkv, wmq, bkv, bmq):
    nblk = N // PROJ_BLK
    full = lambda i: (0, 0)
    return pl.pallas_call(
        _proj_body,
        grid=(nblk,),
        in_specs=[
            pl.BlockSpec((PROJ_BLK, FD), lambda i: (i, 0)),
            pl.BlockSpec((FD, 2 * ED), full),
            pl.BlockSpec((FD, 2 * ED), full),
            pl.BlockSpec((1, 2 * ED), full),
            pl.BlockSpec((1, 2 * ED), full),
        ],
        out_specs=[
            pl.BlockSpec((PROJ_BLK, 2 * ED), lambda i: (i, 0)),
            pl.BlockSpec((PROJ_BLK, 2 * ED), lambda i: (i, 0)),
        ],
        out_shape=[
            jax.ShapeDtypeStruct((N, 2 * ED), jnp.float32),
            jax.ShapeDtypeStruct((N, 2 * ED), jnp.float32),
        ],
    )(id2feat, wkv, wmq, bkv, bmq)


def _make_sc_gather():
    mesh = plsc.VectorSubcoreMesh(core_axis_name="c", subcore_axis_name="s")

    @functools.partial(
        pl.kernel,
        mesh=mesh,
        out_type=jax.ShapeDtypeStruct((BH * S, 2 * ED), jnp.float32),
        scratch_types=(
            [pltpu.VMEM((EPW, CH), jnp.int32)]
            + [pltpu.VMEM((CH, 2 * ED), jnp.float32) for _ in range(NBUF)]
            + [pltpu.SemaphoreType.DMA for _ in range(2 * NBUF)]
        ),
    )
    def sc_gather(tab, idx2d, out, *scratch):
        idx_all = scratch[0]
        rows = scratch[1:1 + NBUF]
        sem_g = scratch[1 + NBUF:1 + 2 * NBUF]
        sem_w = scratch[1 + 2 * NBUF:]
        w = lax.axis_index("c") * NS + lax.axis_index("s")
        lo = w * EPW
        cnt = jnp.clip(NCH - lo, 0, EPW)
        pltpu.sync_copy(idx2d.at[pl.ds(lo, EPW)], idx_all)
        ngrp = EPW // NBUF
        kmax = ngrp * NBUF

        def grp(it, carry):
            for b in range(NBUF):
                k = it * NBUF + b

                @pl.when(jnp.logical_and(k >= NBUF, k - NBUF < cnt))
                def _():
                    pltpu.make_async_copy(
                        out.at[pl.ds(0, CH)], rows[b], sem_w[b]).wait()

                @pl.when(k < cnt)
                def _():
                    pltpu.async_copy(tab.at[idx_all.at[k]], rows[b], sem_g[b])
            for b in range(NBUF):
                k = it * NBUF + b

                @pl.when(k < cnt)
                def _():
                    pltpu.make_async_copy(
                        tab.at[idx_all.at[k]], rows[b], sem_g[b]).wait()
                    pltpu.async_copy(
                        rows[b], out.at[pl.ds((lo + k) * CH, CH)], sem_w[b])
            return carry

        lax.fori_loop(0, ngrp, grp, 0)
        for b in range(NBUF):
            @pl.when(kmax - NBUF + b < cnt)
            def _():
                pltpu.make_async_copy(
                    out.at[pl.ds(0, CH)], rows[b], sem_w[b]).wait()

    return sc_gather


_sc_gather = _make_sc_gather()


def _make_sc_q():
    mesh = plsc.VectorSubcoreMesh(core_axis_name="c", subcore_axis_name="s")

    @functools.partial(
        pl.kernel,
        mesh=mesh,
        out_type=jax.ShapeDtypeStruct((BH, 2 * ED), jnp.float32),
        scratch_types=(
            [pltpu.VMEM((QPW, QCH), jnp.int32)]
            + [pltpu.VMEM((QCH, 2 * ED), jnp.float32) for _ in range(QBUF)]
            + [pltpu.SemaphoreType.DMA for _ in range(2 * QBUF)]
        ),
    )
    def sc_q(pmq, nod_i, qg, *scratch):
        idx_q = scratch[0]
        rows_q = scratch[1:1 + QBUF]
        sem_g = scratch[1 + QBUF:1 + 2 * QBUF]
        sem_w = scratch[1 + 2 * QBUF:]
        w = lax.axis_index("c") * NS + lax.axis_index("s")
        qlo = jnp.minimum(w * QPW, NQCHP - QPW)
        qcnt = jnp.clip(NQCH - w * QPW, 0, QPW)
        pltpu.sync_copy(nod_i.at[pl.ds(qlo, QPW)], idx_q)
        qngrp = QPW // QBUF
        qkmax = qngrp * QBUF

        def qgrp(it, carry):
            for b in range(QBUF):
                k = it * QBUF + b

                @pl.when(jnp.logical_and(k >= QBUF, k - QBUF < qcnt))
                def _():
                    pltpu.make_async_copy(
                        qg.at[pl.ds(0, QCH)], rows_q[b], sem_w[b]).wait()

                @pl.when(k < qcnt)
                def _():
                    pltpu.async_copy(pmq.at[idx_q.at[k]], rows_q[b], sem_g[b])
            for b in range(QBUF):
                k = it * QBUF + b

                @pl.when(k < qcnt)
                def _():
                    pltpu.make_async_copy(
                        pmq.at[idx_q.at[k]], rows_q[b], sem_g[b]).wait()
                    pltpu.async_copy(
                        rows_q[b], qg.at[pl.ds((qlo + k) * QCH, QCH)], sem_w[b])
            return carry

        lax.fori_loop(0, qngrp, qgrp, 0)
        for b in range(QBUF):
            @pl.when(qkmax - QBUF + b < qcnt)
            def _():
                pltpu.make_async_copy(
                    qg.at[pl.ds(0, QCH)], rows_q[b], sem_w[b]).wait()

    return sc_q


_sc_q = _make_sc_q()


def _attn_body(kvg_ref, mg_ref, qg_ref, wc_ref, bc_ref, out_ref):
    kv = kvg_ref[:].reshape(BB, S, 2 * ED)
    k3 = kv[:, :, :ED]
    v3 = kv[:, :, ED:]
    q = qg_ref[:, ED:]
    logits = jnp.sum(k3 * q[:, None, :], axis=-1)
    m = jnp.max(logits, axis=-1, keepdims=True)
    e = jnp.exp(logits - m)
    attn = e / jnp.sum(e, axis=-1, keepdims=True)
    mix = jnp.sum(v3 * attn[:, :, None], axis=1)
    ctx = jnp.sum(mg_ref[:].reshape(BB, S, 2 * ED)[:, :, :ED], axis=1)
    comb = jnp.concatenate([mix, ctx], axis=-1)
    comb = jnp.tanh(jnp.dot(comb, wc_ref[:], preferred_element_type=jnp.float32)
                    + bc_ref[:])
    nrm = jnp.sqrt(jnp.sum(comb * comb, axis=-1, keepdims=True))
    out_ref[:] = comb / jnp.maximum(nrm, 1e-12)


def _attention(kvg, mqg, qg, wc, bc):
    nblk = BH // BB
    return pl.pallas_call(
        _attn_body,
        grid=(nblk,),
        in_specs=[
            pl.BlockSpec((BB * S, 2 * ED), lambda i: (i, 0)),
            pl.BlockSpec((BB * S, 2 * ED), lambda i: (i, 0)),
            pl.BlockSpec((BB, 2 * ED), lambda i: (i, 0)),
            pl.BlockSpec((2 * ED, ED), lambda i: (0, 0)),
            pl.BlockSpec((1, ED), lambda i: (0, 0)),
        ],
        out_specs=pl.BlockSpec((BB, ED), lambda i: (i, 0)),
        out_shape=jax.ShapeDtypeStruct((BH, ED), jnp.float32),
    )(kvg, mqg, qg, wc, bc)


def kernel(id2feat, nodes, adj_neighs, dis_neighs, Wm_w, Wm_b, Wq_w, Wq_b,
           Wk_w, Wk_b, Wv_w, Wv_b, WC_w, WC_b):
    wkv = jnp.concatenate([Wk_w, Wv_w], axis=1)
    bkv = jnp.concatenate([Wk_b, Wv_b])[None, :]
    wmq = jnp.concatenate([Wm_w / S, Wq_w], axis=1)
    bmq = jnp.concatenate([Wm_b / S, Wq_b])[None, :]
    pkv, pmq = _project(id2feat, wkv, wmq, bkv, bmq)
    adj = adj_neighs.astype(jnp.int32)
    dis = dis_neighs.astype(jnp.int32)
    nod = nodes.astype(jnp.int32)
    epad = jnp.zeros((NCHP - NCH, CH), jnp.int32)
    qpad = jnp.zeros((NQCHP - NQCH, QCH), jnp.int32)

    outs = []
    for h in range(NH):
        adj2 = jnp.concatenate(
            [adj[h * BH:(h + 1) * BH].reshape(NCH, CH), epad])
        dis2 = jnp.concatenate(
            [dis[h * BH:(h + 1) * BH].reshape(NCH, CH), epad])
        nod2 = jnp.concatenate(
            [nod[h * BH:(h + 1) * BH].reshape(NQCH, QCH), qpad])
        kvg = _sc_gather(pkv, adj2)
        mqg = _sc_gather(pmq, dis2)
        qg = _sc_q(pmq, nod2)
        outs.append(_attention(kvg, mqg, qg, WC_w, WC_b[None, :]))
    return jnp.concatenate(outs, axis=0)
